# baseline (speedup 1.0000x reference)
import functools

import jax
import jax.numpy as jnp
from jax import lax
from jax.experimental import pallas as pl
from jax.experimental.pallas import tpu as pltpu
from jax.experimental.pallas import tpu_sc as plsc

V = 1000000
D = 64
B = 16384
NEG = 20

DP = 128
NC = 2
NS = 16
NW = NC * NS
BPW = B // NW
CH = 32
NCHUNK = BPW // CH
NEG_ROWS = CH * NEG
NEG_G = NEG_ROWS // 128

_LOG1P_C = (
    9.473308e-10, 0.99999976, -0.4999906, 0.33318192, -0.24872053,
    0.1935175, -0.14533964, 0.09475556, -0.047051135, 0.015055349,
    -0.0022609953,
)


def _perm16(x, idx):
    return lax.gather(
        x, idx[:, None],
        lax.GatherDimensionNumbers(
            offset_dims=(), collapsed_slice_dims=(0,), start_index_map=(0,)),
        slice_sizes=(1,),
        mode=lax.GatherScatterMode.PROMISE_IN_BOUNDS)


def _hsum16(x):
    lanes = lax.iota(jnp.int32, 16)
    for sh in (8, 4, 2, 1):
        x = x + _perm16(x, jnp.bitwise_xor(lanes, sh))
    return x


def _log_sigmoid_vec(x):
    w = jnp.exp(-jnp.abs(x))
    p = jnp.full((16,), _LOG1P_C[-1], jnp.float32)
    for c in _LOG1P_C[-2::-1]:
        p = p * w + c
    return jnp.minimum(x, 0.0) - p


@functools.partial(
    pl.kernel,
    out_type=jax.ShapeDtypeStruct((NW * 16,), jnp.float32),
    mesh=plsc.VectorSubcoreMesh(core_axis_name="c", subcore_axis_name="s"),
    compiler_params=pltpu.CompilerParams(use_tc_tiling_on_sc=True),
    scratch_types=[
        pltpu.VMEM((NCHUNK, CH), jnp.int32),
        pltpu.VMEM((NCHUNK, CH), jnp.int32),
        pltpu.VMEM((NCHUNK * NEG_G, 128), jnp.int32),
        pltpu.VMEM((CH, DP), jnp.float32),
        pltpu.VMEM((CH, DP), jnp.float32),
        pltpu.VMEM((NEG_ROWS, DP), jnp.float32),
        pltpu.VMEM((16,), jnp.float32),
        pltpu.SemaphoreType.DMA,
    ],
)
def _sgns_body(t_hbm, c_hbm, n_hbm, tab_hbm, out_hbm,
               t_idx, c_idx, n_idx, v_rows, u_rows, n_rows, loss_v, sem):
    wid = lax.axis_index("s") * NC + lax.axis_index("c")

    pltpu.sync_copy(t_hbm.at[pl.ds(wid * NCHUNK, NCHUNK)], t_idx)
    pltpu.sync_copy(c_hbm.at[pl.ds(wid * NCHUNK, NCHUNK)], c_idx)
    pltpu.sync_copy(n_hbm.at[pl.ds(wid * NCHUNK * NEG_G, NCHUNK * NEG_G)], n_idx)

    def chunk_body(ch, lv):
        cps = [
            pltpu.async_copy(tab_hbm.at[t_idx.at[ch]], v_rows, sem),
            pltpu.async_copy(tab_hbm.at[c_idx.at[ch]], u_rows, sem),
        ]
        for g in range(NEG_G):
            cps.append(pltpu.async_copy(
                tab_hbm.at[n_idx.at[ch * NEG_G + g]],
                n_rows.at[pl.ds(g * 128, 128)], sem))
        for cp in cps:
            cp.wait()

        def elem_body(e, lv2):
            v = [v_rows[e, pl.ds(16 * k, 16)] for k in range(4)]
            u = [u_rows[e, pl.ds(D + 16 * k, 16)] for k in range(4)]
            pos = v[0] * u[0] + v[1] * u[1] + v[2] * u[2] + v[3] * u[3]
            base = e * NEG
            s = [n_rows[base, pl.ds(D + 16 * k, 16)] for k in range(4)]
            for n in range(1, NEG):
                for k in range(4):
                    s[k] = s[k] + n_rows[base + n, pl.ds(D + 16 * k, 16)]
            nd = s[0] * v[0] + s[1] * v[1] + s[2] * v[2] + s[3] * v[3]
            return (lv2
                    + _log_sigmoid_vec(_hsum16(pos))
                    + _log_sigmoid_vec(-_hsum16(nd)))

        return lax.fori_loop(0, CH, elem_body, lv)

    lv = lax.fori_loop(0, NCHUNK, chunk_body, jnp.zeros((16,), jnp.float32))
    loss_v[...] = lv
    pltpu.sync_copy(loss_v, out_hbm.at[pl.ds(wid * 16, 16)])


_BK = 2048
_TGRID = -(-V // _BK)


def _fuse_body(twt_ref, cwt_ref, out_ref):
    eye = (lax.broadcasted_iota(jnp.int32, (D, D), 0)
           == lax.broadcasted_iota(jnp.int32, (D, D), 1)).astype(jnp.float32)
    dn = (((0,), (0,)), ((), ()))
    t_t = lax.dot_general(twt_ref[...], eye, dn,
                          preferred_element_type=jnp.float32)
    c_t = lax.dot_general(cwt_ref[...], eye, dn,
                          preferred_element_type=jnp.float32)
    out_ref[...] = jnp.concatenate([t_t, c_t], axis=1)


_fuse_tables = pl.pallas_call(
    _fuse_body,
    grid=(_TGRID,),
    in_specs=[
        pl.BlockSpec((D, _BK), lambda j: (0, j)),
        pl.BlockSpec((D, _BK), lambda j: (0, j)),
    ],
    out_specs=pl.BlockSpec((_BK, DP), lambda j: (j, 0)),
    out_shape=jax.ShapeDtypeStruct((V, DP), jnp.float32),
)


def kernel(target_input, context, neg, target_w, context_w):
    t2 = target_input.reshape(NW * NCHUNK, CH)
    c2 = context.reshape(NW * NCHUNK, CH)
    n2 = neg.reshape(NW * NCHUNK * NEG_G, 128)
    tab = _fuse_tables(target_w.T, context_w.T)
    out = _sgns_body(t2, c2, n2, tab)
    return -jnp.sum(out) / (16.0 * B)


# speedup vs baseline: 1.0301x; 1.0301x over previous
import functools

import jax
import jax.numpy as jnp
from jax import lax
from jax.experimental import pallas as pl
from jax.experimental.pallas import tpu as pltpu
from jax.experimental.pallas import tpu_sc as plsc

V = 1000000
D = 64
B = 16384
NEG = 20

DP = 128
NC = 2
NS = 16
NW = NC * NS
BPW = B // NW
CH = 16
NCHUNK = BPW // CH
NEG_ROWS = CH * NEG
NEG_G = 4
NEG_GR = NEG_ROWS // NEG_G

_LOG1P_C = (
    9.473308e-10, 0.99999976, -0.4999906, 0.33318192, -0.24872053,
    0.1935175, -0.14533964, 0.09475556, -0.047051135, 0.015055349,
    -0.0022609953,
)


def _perm16(x, idx):
    return lax.gather(
        x, idx[:, None],
        lax.GatherDimensionNumbers(
            offset_dims=(), collapsed_slice_dims=(0,), start_index_map=(0,)),
        slice_sizes=(1,),
        mode=lax.GatherScatterMode.PROMISE_IN_BOUNDS)


def _hsum16(x):
    lanes = lax.iota(jnp.int32, 16)
    for sh in (8, 4, 2, 1):
        x = x + _perm16(x, jnp.bitwise_xor(lanes, sh))
    return x


def _log_sigmoid_vec(x):
    w = jnp.exp(-jnp.abs(x))
    p = jnp.full((16,), _LOG1P_C[-1], jnp.float32)
    for c in _LOG1P_C[-2::-1]:
        p = p * w + c
    return jnp.minimum(x, 0.0) - p


@functools.partial(
    pl.kernel,
    out_type=jax.ShapeDtypeStruct((NW * 16,), jnp.float32),
    mesh=plsc.VectorSubcoreMesh(core_axis_name="c", subcore_axis_name="s"),
    compiler_params=pltpu.CompilerParams(use_tc_tiling_on_sc=True),
    scratch_types=[
        pltpu.VMEM((BPW,), jnp.int32),
        pltpu.VMEM((BPW,), jnp.int32),
        pltpu.VMEM((BPW * NEG,), jnp.int32),
        pltpu.VMEM((2, CH, DP), jnp.float32),
        pltpu.VMEM((2, CH, DP), jnp.float32),
        pltpu.VMEM((2, NEG_ROWS, DP), jnp.float32),
        pltpu.VMEM((16,), jnp.float32),
        pltpu.SemaphoreType.DMA,
        pltpu.SemaphoreType.DMA,
    ],
)
def _sgns_body(t_hbm, c_hbm, n_hbm, tab_hbm, out_hbm,
               t_idx, c_idx, n_idx, v_rows, u_rows, n_rows, loss_v,
               sem0, sem1):
    wid = lax.axis_index("s") * NC + lax.axis_index("c")
    sems = (sem0, sem1)

    pltpu.sync_copy(t_hbm.at[pl.ds(wid * BPW, BPW)], t_idx)
    pltpu.sync_copy(c_hbm.at[pl.ds(wid * BPW, BPW)], c_idx)
    pltpu.sync_copy(n_hbm.at[pl.ds(wid * BPW * NEG, BPW * NEG)], n_idx)

    def issue(ch, p):
        sem = sems[p]
        pltpu.async_copy(tab_hbm.at[t_idx.at[pl.ds(ch * CH, CH)]],
                         v_rows.at[p], sem)
        pltpu.async_copy(tab_hbm.at[c_idx.at[pl.ds(ch * CH, CH)]],
                         u_rows.at[p], sem)
        for g in range(NEG_G):
            pltpu.async_copy(
                tab_hbm.at[n_idx.at[pl.ds(ch * NEG_ROWS + g * NEG_GR, NEG_GR)]],
                n_rows.at[p, pl.ds(g * NEG_GR, NEG_GR)], sem)

    def drain(p):
        sem = sems[p]
        pltpu.make_async_copy(tab_hbm.at[pl.ds(0, CH)], v_rows.at[p], sem).wait()
        pltpu.make_async_copy(tab_hbm.at[pl.ds(0, CH)], u_rows.at[p], sem).wait()
        pltpu.make_async_copy(tab_hbm.at[pl.ds(0, NEG_ROWS)], n_rows.at[p],
                              sem).wait()

    def compute(p, lv):
        def elem_body(e, lv2):
            v = [v_rows[p, e, pl.ds(16 * k, 16)] for k in range(4)]
            u = [u_rows[p, e, pl.ds(D + 16 * k, 16)] for k in range(4)]
            pos = v[0] * u[0] + v[1] * u[1] + v[2] * u[2] + v[3] * u[3]
            base = e * NEG
            s = [n_rows[p, base, pl.ds(D + 16 * k, 16)] for k in range(4)]
            for n in range(1, NEG):
                for k in range(4):
                    s[k] = s[k] + n_rows[p, base + n, pl.ds(D + 16 * k, 16)]
            nd = s[0] * v[0] + s[1] * v[1] + s[2] * v[2] + s[3] * v[3]
            return (lv2
                    + _log_sigmoid_vec(_hsum16(pos))
                    + _log_sigmoid_vec(-_hsum16(nd)))

        return lax.fori_loop(0, CH, elem_body, lv)

    issue(0, 0)

    def pair_body(i, lv):
        ch = i * 2
        issue(ch + 1, 1)
        drain(0)
        lv = compute(0, lv)

        @pl.when(ch + 2 < NCHUNK)
        def _():
            issue(ch + 2, 0)

        drain(1)
        return compute(1, lv)

    lv = lax.fori_loop(0, NCHUNK // 2, pair_body,
                       jnp.zeros((16,), jnp.float32))
    loss_v[...] = lv
    pltpu.sync_copy(loss_v, out_hbm.at[pl.ds(wid * 16, 16)])


_BK = 2048
_TGRID = -(-V // _BK)


def _fuse_body(twt_ref, cwt_ref, out_ref):
    eye = (lax.broadcasted_iota(jnp.int32, (D, D), 0)
           == lax.broadcasted_iota(jnp.int32, (D, D), 1)).astype(jnp.float32)
    dn = (((0,), (0,)), ((), ()))
    t_t = lax.dot_general(twt_ref[...], eye, dn,
                          preferred_element_type=jnp.float32)
    c_t = lax.dot_general(cwt_ref[...], eye, dn,
                          preferred_element_type=jnp.float32)
    out_ref[...] = jnp.concatenate([t_t, c_t], axis=1)


_fuse_tables = pl.pallas_call(
    _fuse_body,
    grid=(_TGRID,),
    in_specs=[
        pl.BlockSpec((D, _BK), lambda j: (0, j)),
        pl.BlockSpec((D, _BK), lambda j: (0, j)),
    ],
    out_specs=pl.BlockSpec((_BK, DP), lambda j: (j, 0)),
    out_shape=jax.ShapeDtypeStruct((V, DP), jnp.float32),
)


def kernel(target_input, context, neg, target_w, context_w):
    t2 = target_input
    c2 = context
    n2 = neg.reshape(B * NEG)
    tab = _fuse_tables(target_w.T, context_w.T)
    out = _sgns_body(t2, c2, n2, tab)
    return -jnp.sum(out) / (16.0 * B)


# speedup vs baseline: 1.2523x; 1.2157x over previous
import functools

import jax
import jax.numpy as jnp
from jax import lax
from jax.experimental import pallas as pl
from jax.experimental.pallas import tpu as pltpu
from jax.experimental.pallas import tpu_sc as plsc

V = 1000000
D = 64
B = 16384
NEG = 20

DP = 128
NC = 2
NS = 16
NW = NC * NS
BPW = B // NW
CH = 16
NCHUNK = BPW // CH
NEG_ROWS = CH * NEG
NEG_G = 4
NEG_GR = NEG_ROWS // NEG_G

_LOG1P_C = (
    9.473308e-10, 0.99999976, -0.4999906, 0.33318192, -0.24872053,
    0.1935175, -0.14533964, 0.09475556, -0.047051135, 0.015055349,
    -0.0022609953,
)


def _perm16(x, idx):
    return lax.gather(
        x, idx[:, None],
        lax.GatherDimensionNumbers(
            offset_dims=(), collapsed_slice_dims=(0,), start_index_map=(0,)),
        slice_sizes=(1,),
        mode=lax.GatherScatterMode.PROMISE_IN_BOUNDS)


def _hsum16(x):
    lanes = lax.iota(jnp.int32, 16)
    for sh in (8, 4, 2, 1):
        x = x + _perm16(x, jnp.bitwise_xor(lanes, sh))
    return x


def _log_sigmoid_vec(x):
    w = jnp.exp(-jnp.abs(x))
    p = jnp.full((16,), _LOG1P_C[-1], jnp.float32)
    for c in _LOG1P_C[-2::-1]:
        p = p * w + c
    return jnp.minimum(x, 0.0) - p


@functools.partial(
    pl.kernel,
    out_type=jax.ShapeDtypeStruct((NW * 16,), jnp.float32),
    mesh=plsc.VectorSubcoreMesh(core_axis_name="c", subcore_axis_name="s"),
    compiler_params=pltpu.CompilerParams(use_tc_tiling_on_sc=True),
    scratch_types=[
        pltpu.VMEM((BPW,), jnp.int32),
        pltpu.VMEM((BPW,), jnp.int32),
        pltpu.VMEM((BPW * NEG,), jnp.int32),
        pltpu.VMEM((2, CH, DP), jnp.float32),
        pltpu.VMEM((2, CH, DP), jnp.float32),
        pltpu.VMEM((2, NEG_ROWS, DP), jnp.float32),
        pltpu.VMEM((16,), jnp.float32),
        pltpu.SemaphoreType.DMA,
        pltpu.SemaphoreType.DMA,
    ],
)
def _sgns_body(t_hbm, c_hbm, n_hbm, tab_hbm, out_hbm,
               t_idx, c_idx, n_idx, v_rows, u_rows, n_rows, loss_v,
               sem0, sem1):
    wid = lax.axis_index("s") * NC + lax.axis_index("c")
    sems = (sem0, sem1)

    pltpu.sync_copy(t_hbm.at[pl.ds(wid * BPW, BPW)], t_idx)
    pltpu.sync_copy(c_hbm.at[pl.ds(wid * BPW, BPW)], c_idx)
    pltpu.sync_copy(n_hbm.at[pl.ds(wid * BPW * NEG, BPW * NEG)], n_idx)

    def issue(ch, p):
        sem = sems[p]
        pltpu.async_copy(tab_hbm.at[t_idx.at[pl.ds(ch * CH, CH)]],
                         v_rows.at[p], sem)
        pltpu.async_copy(tab_hbm.at[c_idx.at[pl.ds(ch * CH, CH)]],
                         u_rows.at[p], sem)
        for g in range(NEG_G):
            pltpu.async_copy(
                tab_hbm.at[n_idx.at[pl.ds(ch * NEG_ROWS + g * NEG_GR, NEG_GR)]],
                n_rows.at[p, pl.ds(g * NEG_GR, NEG_GR)], sem)

    def drain(p):
        sem = sems[p]
        pltpu.make_async_copy(tab_hbm.at[pl.ds(0, CH)], v_rows.at[p], sem).wait()
        pltpu.make_async_copy(tab_hbm.at[pl.ds(0, CH)], u_rows.at[p], sem).wait()
        pltpu.make_async_copy(tab_hbm.at[pl.ds(0, NEG_ROWS)], n_rows.at[p],
                              sem).wait()

    def compute(p, lv):
        def elem_body(e, lv2):
            v = [v_rows[p, e, pl.ds(16 * k, 16)] for k in range(4)]
            u = [u_rows[p, e, pl.ds(D + 16 * k, 16)] for k in range(4)]
            pos = v[0] * u[0] + v[1] * u[1] + v[2] * u[2] + v[3] * u[3]
            base = e * NEG
            s = [n_rows[p, base, pl.ds(D + 16 * k, 16)] for k in range(4)]
            for n in range(1, NEG):
                for k in range(4):
                    s[k] = s[k] + n_rows[p, base + n, pl.ds(D + 16 * k, 16)]
            nd = s[0] * v[0] + s[1] * v[1] + s[2] * v[2] + s[3] * v[3]
            return (lv2
                    + _log_sigmoid_vec(_hsum16(pos))
                    + _log_sigmoid_vec(-_hsum16(nd)))

        return lax.fori_loop(0, CH, elem_body, lv)

    issue(0, 0)

    def pair_body(i, lv):
        ch = i * 2
        issue(ch + 1, 1)
        drain(0)
        lv = compute(0, lv)

        @pl.when(ch + 2 < NCHUNK)
        def _():
            issue(ch + 2, 0)

        drain(1)
        return compute(1, lv)

    lv = lax.fori_loop(0, NCHUNK // 2, pair_body,
                       jnp.zeros((16,), jnp.float32))
    loss_v[...] = lv
    pltpu.sync_copy(loss_v, out_hbm.at[pl.ds(wid * 16, 16)])


_BK = 4096
_TGRID = -(-V // _BK)


def _fuse_body(twt_ref, cwt_ref, out_ref):
    eye = (lax.broadcasted_iota(jnp.int32, (D, D), 0)
           == lax.broadcasted_iota(jnp.int32, (D, D), 1)).astype(jnp.float32)
    dn = (((0,), (0,)), ((), ()))
    t_t = lax.dot_general(twt_ref[...], eye, dn,
                          preferred_element_type=jnp.float32)
    c_t = lax.dot_general(cwt_ref[...], eye, dn,
                          preferred_element_type=jnp.float32)
    out_ref[...] = jnp.concatenate([t_t, c_t], axis=1)


_fuse_tables = pl.pallas_call(
    _fuse_body,
    grid=(_TGRID,),
    in_specs=[
        pl.BlockSpec((D, _BK), lambda j: (0, j)),
        pl.BlockSpec((D, _BK), lambda j: (0, j)),
    ],
    out_specs=pl.BlockSpec((_BK, DP), lambda j: (j, 0)),
    out_shape=jax.ShapeDtypeStruct((V, DP), jnp.float32),
)


def kernel(target_input, context, neg, target_w, context_w):
    t2 = target_input
    c2 = context
    n2 = neg.reshape(B * NEG)
    tab = _fuse_tables(target_w.T, context_w.T)
    out = _sgns_body(t2, c2, n2, tab)
    return -jnp.sum(out) / (16.0 * B)


# speedup vs baseline: 1.4067x; 1.1233x over previous
import functools

import jax
import jax.numpy as jnp
from jax import lax
from jax.experimental import pallas as pl
from jax.experimental.pallas import tpu as pltpu
from jax.experimental.pallas import tpu_sc as plsc

V = 1000000
D = 64
B = 16384
NEG = 20

DP = 128
NC = 2
NS = 16
NW = NC * NS
BPW = B // NW
CH = 16
NCHUNK = BPW // CH
NEG_ROWS = CH * NEG
NEG_G = 4
NEG_GR = NEG_ROWS // NEG_G

_LOG1P_C = (
    9.473308e-10, 0.99999976, -0.4999906, 0.33318192, -0.24872053,
    0.1935175, -0.14533964, 0.09475556, -0.047051135, 0.015055349,
    -0.0022609953,
)


def _perm16(x, idx):
    return lax.gather(
        x, idx[:, None],
        lax.GatherDimensionNumbers(
            offset_dims=(), collapsed_slice_dims=(0,), start_index_map=(0,)),
        slice_sizes=(1,),
        mode=lax.GatherScatterMode.PROMISE_IN_BOUNDS)


def _hsum16(x):
    lanes = lax.iota(jnp.int32, 16)
    for sh in (8, 4, 2, 1):
        x = x + _perm16(x, jnp.bitwise_xor(lanes, sh))
    return x


def _log_sigmoid_vec(x):
    w = jnp.exp(-jnp.abs(x))
    p = jnp.full((16,), _LOG1P_C[-1], jnp.float32)
    for c in _LOG1P_C[-2::-1]:
        p = p * w + c
    return jnp.minimum(x, 0.0) - p


@functools.partial(
    pl.kernel,
    out_type=jax.ShapeDtypeStruct((NW * 16,), jnp.float32),
    mesh=plsc.VectorSubcoreMesh(core_axis_name="c", subcore_axis_name="s"),
    compiler_params=pltpu.CompilerParams(use_tc_tiling_on_sc=True),
    scratch_types=[
        pltpu.VMEM((BPW,), jnp.int32),
        pltpu.VMEM((BPW,), jnp.int32),
        pltpu.VMEM((BPW * NEG,), jnp.int32),
        pltpu.VMEM((2, CH, DP), jnp.float32),
        pltpu.VMEM((2, CH, DP), jnp.float32),
        pltpu.VMEM((2, NEG_ROWS, DP), jnp.float32),
        pltpu.VMEM((16,), jnp.float32),
        pltpu.SemaphoreType.DMA,
        pltpu.SemaphoreType.DMA,
    ],
)
def _sgns_body(t_hbm, c_hbm, n_hbm, tab_hbm, out_hbm,
               t_idx, c_idx, n_idx, v_rows, u_rows, n_rows, loss_v,
               sem0, sem1):
    wid = lax.axis_index("s") * NC + lax.axis_index("c")
    sems = (sem0, sem1)

    pltpu.sync_copy(t_hbm.at[pl.ds(wid * BPW, BPW)], t_idx)
    pltpu.sync_copy(c_hbm.at[pl.ds(wid * BPW, BPW)], c_idx)
    pltpu.sync_copy(n_hbm.at[pl.ds(wid * BPW * NEG, BPW * NEG)], n_idx)

    def issue(ch, p):
        sem = sems[p]
        pltpu.async_copy(tab_hbm.at[t_idx.at[pl.ds(ch * CH, CH)]],
                         v_rows.at[p], sem)
        pltpu.async_copy(tab_hbm.at[c_idx.at[pl.ds(ch * CH, CH)]],
                         u_rows.at[p], sem)
        for g in range(NEG_G):
            pltpu.async_copy(
                tab_hbm.at[n_idx.at[pl.ds(ch * NEG_ROWS + g * NEG_GR, NEG_GR)]],
                n_rows.at[p, pl.ds(g * NEG_GR, NEG_GR)], sem)

    def drain(p):
        sem = sems[p]
        pltpu.make_async_copy(tab_hbm.at[pl.ds(0, CH)], v_rows.at[p], sem).wait()
        pltpu.make_async_copy(tab_hbm.at[pl.ds(0, CH)], u_rows.at[p], sem).wait()
        pltpu.make_async_copy(tab_hbm.at[pl.ds(0, NEG_ROWS)], n_rows.at[p],
                              sem).wait()

    def compute(p, lv):
        def elem_body(e, lv2):
            v = [v_rows[p, e, pl.ds(16 * k, 16)] for k in range(4)]
            u = [u_rows[p, e, pl.ds(D + 16 * k, 16)] for k in range(4)]
            pos = v[0] * u[0] + v[1] * u[1] + v[2] * u[2] + v[3] * u[3]
            base = e * NEG
            s = [n_rows[p, base, pl.ds(D + 16 * k, 16)] for k in range(4)]
            for n in range(1, NEG):
                for k in range(4):
                    s[k] = s[k] + n_rows[p, base + n, pl.ds(D + 16 * k, 16)]
            nd = s[0] * v[0] + s[1] * v[1] + s[2] * v[2] + s[3] * v[3]
            return (lv2
                    + _log_sigmoid_vec(_hsum16(pos))
                    + _log_sigmoid_vec(-_hsum16(nd)))

        return lax.fori_loop(0, CH, elem_body, lv)

    issue(0, 0)

    def pair_body(i, lv):
        ch = i * 2
        issue(ch + 1, 1)
        drain(0)
        lv = compute(0, lv)

        @pl.when(ch + 2 < NCHUNK)
        def _():
            issue(ch + 2, 0)

        drain(1)
        return compute(1, lv)

    lv = lax.fori_loop(0, NCHUNK // 2, pair_body,
                       jnp.zeros((16,), jnp.float32))
    loss_v[...] = lv
    pltpu.sync_copy(loss_v, out_hbm.at[pl.ds(wid * 16, 16)])


_BK = 8192
_TGRID = -(-V // _BK)


def _fuse_body(twt_ref, cwt_ref, out_ref):
    eye = (lax.broadcasted_iota(jnp.int32, (D, D), 0)
           == lax.broadcasted_iota(jnp.int32, (D, D), 1)).astype(jnp.float32)
    dn = (((0,), (0,)), ((), ()))
    t_t = lax.dot_general(twt_ref[...], eye, dn,
                          preferred_element_type=jnp.float32)
    c_t = lax.dot_general(cwt_ref[...], eye, dn,
                          preferred_element_type=jnp.float32)
    out_ref[...] = jnp.concatenate([t_t, c_t], axis=1)


_fuse_tables = pl.pallas_call(
    _fuse_body,
    grid=(_TGRID,),
    in_specs=[
        pl.BlockSpec((D, _BK), lambda j: (0, j)),
        pl.BlockSpec((D, _BK), lambda j: (0, j)),
    ],
    out_specs=pl.BlockSpec((_BK, DP), lambda j: (j, 0)),
    out_shape=jax.ShapeDtypeStruct((V, DP), jnp.float32),
)


def kernel(target_input, context, neg, target_w, context_w):
    t2 = target_input
    c2 = context
    n2 = neg.reshape(B * NEG)
    tab = _fuse_tables(target_w.T, context_w.T)
    out = _sgns_body(t2, c2, n2, tab)
    return -jnp.sum(out) / (16.0 * B)


# speedup vs baseline: 1.4874x; 1.0574x over previous
import functools

import jax
import jax.numpy as jnp
from jax import lax
from jax.experimental import pallas as pl
from jax.experimental.pallas import tpu as pltpu
from jax.experimental.pallas import tpu_sc as plsc

V = 1000000
D = 64
B = 16384
NEG = 20

DP = 128
NC = 2
NS = 16
NW = NC * NS
BPW = B // NW
CH = 16
NCHUNK = BPW // CH
NEG_ROWS = CH * NEG
NEG_G = 4
NEG_GR = NEG_ROWS // NEG_G

_LOG1P_C = (
    9.473308e-10, 0.99999976, -0.4999906, 0.33318192, -0.24872053,
    0.1935175, -0.14533964, 0.09475556, -0.047051135, 0.015055349,
    -0.0022609953,
)


def _perm16(x, idx):
    return lax.gather(
        x, idx[:, None],
        lax.GatherDimensionNumbers(
            offset_dims=(), collapsed_slice_dims=(0,), start_index_map=(0,)),
        slice_sizes=(1,),
        mode=lax.GatherScatterMode.PROMISE_IN_BOUNDS)


def _hsum16(x):
    lanes = lax.iota(jnp.int32, 16)
    for sh in (8, 4, 2, 1):
        x = x + _perm16(x, jnp.bitwise_xor(lanes, sh))
    return x


def _log_sigmoid_vec(x):
    w = jnp.exp(-jnp.abs(x))
    p = jnp.full((16,), _LOG1P_C[-1], jnp.float32)
    for c in _LOG1P_C[-2::-1]:
        p = p * w + c
    return jnp.minimum(x, 0.0) - p


@functools.partial(
    pl.kernel,
    out_type=jax.ShapeDtypeStruct((NW * 16,), jnp.float32),
    mesh=plsc.VectorSubcoreMesh(core_axis_name="c", subcore_axis_name="s"),
    compiler_params=pltpu.CompilerParams(use_tc_tiling_on_sc=True),
    scratch_types=[
        pltpu.VMEM((BPW,), jnp.int32),
        pltpu.VMEM((BPW,), jnp.int32),
        pltpu.VMEM((BPW * NEG,), jnp.int32),
        pltpu.VMEM((2, CH, DP), jnp.float32),
        pltpu.VMEM((2, CH, DP), jnp.float32),
        pltpu.VMEM((2, NEG_ROWS, DP), jnp.float32),
        pltpu.VMEM((16,), jnp.float32),
        pltpu.SemaphoreType.DMA,
        pltpu.SemaphoreType.DMA,
    ],
)
def _sgns_body(t_hbm, c_hbm, n_hbm, tab_hbm, out_hbm,
               t_idx, c_idx, n_idx, v_rows, u_rows, n_rows, loss_v,
               sem0, sem1):
    wid = lax.axis_index("s") * NC + lax.axis_index("c")
    sems = (sem0, sem1)

    pltpu.sync_copy(t_hbm.at[pl.ds(wid * BPW, BPW)], t_idx)
    pltpu.sync_copy(c_hbm.at[pl.ds(wid * BPW, BPW)], c_idx)
    pltpu.sync_copy(n_hbm.at[pl.ds(wid * BPW * NEG, BPW * NEG)], n_idx)

    def issue(ch, p):
        sem = sems[p]
        pltpu.async_copy(tab_hbm.at[t_idx.at[pl.ds(ch * CH, CH)]],
                         v_rows.at[p], sem)
        pltpu.async_copy(tab_hbm.at[c_idx.at[pl.ds(ch * CH, CH)]],
                         u_rows.at[p], sem)
        for g in range(NEG_G):
            pltpu.async_copy(
                tab_hbm.at[n_idx.at[pl.ds(ch * NEG_ROWS + g * NEG_GR, NEG_GR)]],
                n_rows.at[p, pl.ds(g * NEG_GR, NEG_GR)], sem)

    def drain(p):
        sem = sems[p]
        pltpu.make_async_copy(tab_hbm.at[pl.ds(0, CH)], v_rows.at[p], sem).wait()
        pltpu.make_async_copy(tab_hbm.at[pl.ds(0, CH)], u_rows.at[p], sem).wait()
        pltpu.make_async_copy(tab_hbm.at[pl.ds(0, NEG_ROWS)], n_rows.at[p],
                              sem).wait()

    def compute(p, lv):
        def elem_body(e, lv2):
            v = [v_rows[p, e, pl.ds(16 * k, 16)] for k in range(4)]
            u = [u_rows[p, e, pl.ds(D + 16 * k, 16)] for k in range(4)]
            pos = v[0] * u[0] + v[1] * u[1] + v[2] * u[2] + v[3] * u[3]
            base = e * NEG
            s = [n_rows[p, base, pl.ds(D + 16 * k, 16)] for k in range(4)]
            for n in range(1, NEG):
                for k in range(4):
                    s[k] = s[k] + n_rows[p, base + n, pl.ds(D + 16 * k, 16)]
            nd = s[0] * v[0] + s[1] * v[1] + s[2] * v[2] + s[3] * v[3]
            return (lv2
                    + _log_sigmoid_vec(_hsum16(pos))
                    + _log_sigmoid_vec(-_hsum16(nd)))

        return lax.fori_loop(0, CH, elem_body, lv)

    issue(0, 0)

    def pair_body(i, lv):
        ch = i * 2
        issue(ch + 1, 1)
        drain(0)
        lv = compute(0, lv)

        @pl.when(ch + 2 < NCHUNK)
        def _():
            issue(ch + 2, 0)

        drain(1)
        return compute(1, lv)

    lv = lax.fori_loop(0, NCHUNK // 2, pair_body,
                       jnp.zeros((16,), jnp.float32))
    loss_v[...] = lv
    pltpu.sync_copy(loss_v, out_hbm.at[pl.ds(wid * 16, 16)])


_BK = 16384
_TGRID = -(-V // _BK)


def _fuse_body(twt_ref, cwt_ref, out_ref):
    eye = (lax.broadcasted_iota(jnp.int32, (D, D), 0)
           == lax.broadcasted_iota(jnp.int32, (D, D), 1)).astype(jnp.float32)
    dn = (((0,), (0,)), ((), ()))
    t_t = lax.dot_general(twt_ref[...], eye, dn,
                          preferred_element_type=jnp.float32)
    c_t = lax.dot_general(cwt_ref[...], eye, dn,
                          preferred_element_type=jnp.float32)
    out_ref[...] = jnp.concatenate([t_t, c_t], axis=1)


_fuse_tables = pl.pallas_call(
    _fuse_body,
    grid=(_TGRID,),
    in_specs=[
        pl.BlockSpec((D, _BK), lambda j: (0, j)),
        pl.BlockSpec((D, _BK), lambda j: (0, j)),
    ],
    out_specs=pl.BlockSpec((_BK, DP), lambda j: (j, 0)),
    out_shape=jax.ShapeDtypeStruct((V, DP), jnp.float32),
)


def kernel(target_input, context, neg, target_w, context_w):
    t2 = target_input
    c2 = context
    n2 = neg.reshape(B * NEG)
    tab = _fuse_tables(target_w.T, context_w.T)
    out = _sgns_body(t2, c2, n2, tab)
    return -jnp.sum(out) / (16.0 * B)
